# baseline (speedup 1.0000x reference)
import functools

import jax
import jax.numpy as jnp
from jax import lax
from jax.experimental import pallas as pl
from jax.experimental.pallas import tpu as pltpu
from jax.experimental.pallas import tpu_sc as plsc

N = 10000
E = 160000
D = 256
R = 16
H = 128
NR = N * R
NSUB = 16
EP = E // NSUB
G = 80
EPP = 10240
CH = EPP // G
NDUM = 16
BN = 1000
NB = N // BN


def _transform_body(apply_scale, h_ref, scale_ref, w_ref, rf_ref, out_ref):
    h = h_ref[...]
    if apply_scale:
        h = h * scale_ref[...]
    for r in range(R):
        p = jnp.dot(h, w_ref[r], preferred_element_type=jnp.float32)
        p = p + rf_ref[r][None, :]
        out_ref[0, r] = p[:, :H]
        out_ref[1, r] = p[:, H:]


def _transform(h, scale, w, rf, apply_scale):
    return pl.pallas_call(
        functools.partial(_transform_body, apply_scale),
        grid=(NB,),
        in_specs=[
            pl.BlockSpec((BN, D), lambda i: (i, 0)),
            pl.BlockSpec((BN, 1), lambda i: (i, 0)),
            pl.BlockSpec((R, D, D), lambda i: (0, 0, 0)),
            pl.BlockSpec((R, D), lambda i: (0, 0)),
        ],
        out_specs=pl.BlockSpec((2, R, BN, H), lambda i: (0, 0, i, 0)),
        out_shape=jax.ShapeDtypeStruct((2, R, N, H), jnp.float32),
    )(h, scale, w, rf)


@functools.cache
def _make_sc_scatter():
    mesh = plsc.VectorSubcoreMesh(core_axis_name="c", subcore_axis_name="s")
    return functools.partial(
        pl.kernel,
        mesh=mesh,
        out_type=jax.ShapeDtypeStruct((N, D), jnp.float32),
        scratch_types=[
            pltpu.VMEM((4, G, H), jnp.float32),
            pltpu.VMEM((4, G), jnp.int32),
            pltpu.VMEM((4, G), jnp.int32),
            pltpu.VMEM_SHARED((N + NDUM, H), jnp.float32),
            pltpu.SemaphoreType.DMA,
            pltpu.SemaphoreType.DMA,
            pltpu.SemaphoreType.DMA,
            pltpu.SemaphoreType.DMA,
            pltpu.SemaphoreType.DMA,
            pltpu.SemaphoreType.DMA,
            pltpu.SemaphoreType.DMA,
            pltpu.SemaphoreType.DMA,
        ],
    )(_sc_scatter_body)


def _sc_scatter_body(keyp_hbm, dstp_hbm, t_hbm, out_hbm,
                     rows_r, kb_r, db_r, acc,
                     gs0, gs1, gs2, gs3, is0, is1, is2, is3):
    gsems = (gs0, gs1, gs2, gs3)
    isems = (is0, is1, is2, is3)
    c = lax.axis_index("c")
    s = lax.axis_index("s")
    ebase = s * EPP

    def _zero(i, carry):
        r = i // (H // 16)
        col = (i % (H // 16)) * 16
        rows_r[0, r, pl.ds(col, 16)] = jnp.zeros((16,), jnp.float32)
        return carry
    lax.fori_loop(0, G * (H // 16), _zero, 0)

    zbase = s * (N // NSUB)
    for k in range(7):
        pltpu.sync_copy(rows_r.at[0], acc.at[pl.ds(zbase + k * G, G)])
    pltpu.sync_copy(rows_r.at[0, pl.ds(0, 65)], acc.at[pl.ds(zbase + 7 * G, 65)])
    plsc.subcore_barrier()

    kbase = c * (NSUB * EPP) + ebase

    def _fire_idx(chunk, b):
        pltpu.async_copy(keyp_hbm.at[pl.ds(kbase + chunk * G, G)],
                         kb_r.at[b], isems[b])
        pltpu.async_copy(dstp_hbm.at[pl.ds(ebase + chunk * G, G)],
                         db_r.at[b], isems[b])

    def _wait_idx(b):
        pltpu.make_async_copy(dstp_hbm.at[pl.ds(0, G)], kb_r.at[b], isems[b]).wait()
        pltpu.make_async_copy(dstp_hbm.at[pl.ds(0, G)], db_r.at[b], isems[b]).wait()

    def _fire_rows(b):
        pltpu.async_copy(t_hbm.at[kb_r.at[b]], rows_r.at[b], gsems[b])

    def _wait_rows(b):
        pltpu.make_async_copy(t_hbm.at[kb_r.at[b]], rows_r.at[b], gsems[b]).wait()

    _fire_idx(0, 0)
    _fire_idx(1, 1)
    _fire_idx(2, 2)
    _wait_idx(0)
    _fire_rows(0)
    _wait_idx(1)
    _fire_rows(1)

    def _ring(g, carry):
        for b in range(4):
            k = 4 * g + b
            b2 = (b + 2) % 4
            b3 = (b + 3) % 4
            @pl.when(k + 3 < CH)
            def _():
                _fire_idx(k + 3, b3)
            @pl.when(k + 2 < CH)
            def _():
                _wait_idx(b2)
                _fire_rows(b2)
            _wait_rows(b)
            pltpu.sync_copy(rows_r.at[b], acc.at[db_r.at[b]], add=True)
        return carry
    lax.fori_loop(0, CH // 4, _ring, 0)
    plsc.subcore_barrier()

    rbase = s * 624
    pltpu.sync_copy(acc.at[pl.ds(rbase, 624)],
                    out_hbm.at[pl.ds(rbase, 624), pl.ds(c * H, H)])
    @pl.when(s == NSUB - 1)
    def _tail():
        pltpu.sync_copy(acc.at[pl.ds(9984, 16)],
                        out_hbm.at[pl.ds(9984, 16), pl.ds(c * H, H)])


def _final_body(agg_ref, norm_ref, lw_ref, out_ref):
    h2 = agg_ref[...] * norm_ref[...]
    out_ref[...] = jnp.maximum(
        h2 + jnp.dot(h2, lw_ref[...], preferred_element_type=jnp.float32), 0.0)


def _final(agg, normv, loop_weight):
    return pl.pallas_call(
        _final_body,
        grid=(NB,),
        in_specs=[
            pl.BlockSpec((BN, D), lambda i: (i, 0)),
            pl.BlockSpec((BN, 1), lambda i: (i, 0)),
            pl.BlockSpec((D, D), lambda i: (0, 0)),
        ],
        out_specs=pl.BlockSpec((BN, D), lambda i: (i, 0)),
        out_shape=jax.ShapeDtypeStruct((N, D), jnp.float32),
    )(agg, normv, loop_weight)


def kernel(x, norm, edge_index, edge_type, rFeatures, relation_weights, loop_weight):
    src = edge_index[0].astype(jnp.int32)
    dst = edge_index[1].astype(jnp.int32)
    key = edge_type.astype(jnp.int32) * N + src
    normv = norm.reshape(N, 1)

    key2 = jnp.stack([key, key + NR]).reshape(2, NSUB, EP)
    keyp = jnp.pad(key2, ((0, 0), (0, 0), (0, EPP - EP))).reshape(2 * NSUB * EPP)
    dstp = jnp.pad(dst.reshape(NSUB, EP), ((0, 0), (0, EPP - EP)),
                   constant_values=N).reshape(NSUB * EPP)

    sc_scatter = _make_sc_scatter()
    t0 = _transform(x, normv, relation_weights, rFeatures, apply_scale=False)
    agg1 = sc_scatter(keyp, dstp, t0.reshape(2 * NR, H))
    t1 = _transform(agg1, normv, relation_weights, rFeatures, apply_scale=True)
    agg2 = sc_scatter(keyp, dstp, t1.reshape(2 * NR, H))
    return _final(agg2, normv, loop_weight)


# speedup vs baseline: 1.3975x; 1.3975x over previous
import functools

import numpy as np
import jax
import jax.numpy as jnp
from jax import lax
from jax.experimental import pallas as pl
from jax.experimental.pallas import tpu as pltpu
from jax.experimental.pallas import tpu_sc as plsc

N = 10000
E = 160000
D = 256
R = 16
H = 128
NR = N * R
NSUB = 16
EP = E // NSUB
G = 128
EPP = 10240
CH = EPP // G
NDUM = 16
BN = 1000
NB = N // BN

KBITS = 18
KMASK = (1 << KBITS) - 1
_PKPAD = np.int32(np.uint32(N << KBITS).astype(np.int64) - (1 << 32))


def _transform_body(apply_scale, h_ref, scale_ref, w_ref, rf_ref, out_ref):
    h = h_ref[...]
    if apply_scale:
        h = h * scale_ref[...]
    for r in range(R):
        p = jnp.dot(h, w_ref[r], preferred_element_type=jnp.float32)
        p = p + rf_ref[r][None, :]
        out_ref[0, r] = p[:, :H]
        out_ref[1, r] = p[:, H:]


def _transform(h, scale, w, rf, apply_scale):
    return pl.pallas_call(
        functools.partial(_transform_body, apply_scale),
        grid=(NB,),
        in_specs=[
            pl.BlockSpec((BN, D), lambda i: (i, 0)),
            pl.BlockSpec((BN, 1), lambda i: (i, 0)),
            pl.BlockSpec((R, D, D), lambda i: (0, 0, 0)),
            pl.BlockSpec((R, D), lambda i: (0, 0)),
        ],
        out_specs=pl.BlockSpec((2, R, BN, H), lambda i: (0, 0, i, 0)),
        out_shape=jax.ShapeDtypeStruct((2, R, N, H), jnp.float32),
    )(h, scale, w, rf)


@functools.cache
def _make_sc_scatter():
    mesh = plsc.VectorSubcoreMesh(core_axis_name="c", subcore_axis_name="s")
    return functools.partial(
        pl.kernel,
        mesh=mesh,
        out_type=jax.ShapeDtypeStruct((N, D), jnp.float32),
        scratch_types=[
            pltpu.VMEM((EPP,), jnp.int32),
            pltpu.VMEM((2, G, H), jnp.float32),
            pltpu.VMEM((2, G), jnp.int32),
            pltpu.VMEM((2, G), jnp.int32),
            pltpu.VMEM_SHARED((N + NDUM, H), jnp.float32),
            pltpu.SemaphoreType.DMA,
            pltpu.SemaphoreType.DMA,
        ],
    )(_sc_scatter_body)


def _sc_scatter_body(pk_hbm, t_hbm, out_hbm,
                     pk_v, rows_r, kb_r, db_r, acc, sem0, sem1):
    sems = (sem0, sem1)
    c = lax.axis_index("c")
    s = lax.axis_index("s")

    pltpu.sync_copy(pk_hbm.at[pl.ds(s * EP, EP)], pk_v.at[pl.ds(0, EP)])

    def _pad(i, carry):
        pk_v[pl.ds(EP + i * 16, 16)] = jnp.full((16,), _PKPAD, jnp.int32)
        return carry
    lax.fori_loop(0, (EPP - EP) // 16, _pad, 0)

    def _zero(i, carry):
        rows_r[0, i // (H // 16), pl.ds((i % (H // 16)) * 16, 16)] = (
            jnp.zeros((16,), jnp.float32))
        return carry
    lax.fori_loop(0, G * (H // 16), _zero, 0)

    zbase = s * (N // NSUB)
    for k in range(4):
        pltpu.sync_copy(rows_r.at[0], acc.at[pl.ds(zbase + k * G, G)])
    pltpu.sync_copy(rows_r.at[0, pl.ds(0, 113)],
                    acc.at[pl.ds(zbase + 4 * G, 113)])
    plsc.subcore_barrier()

    koff = c * NR

    def _stage_kb(chunk, b):
        cb = chunk * G
        def _cp(j, inner):
            v = pk_v[pl.ds(cb + j * 16, 16)]
            kb_r[b, pl.ds(j * 16, 16)] = (v & KMASK) + koff
            return inner
        lax.fori_loop(0, G // 16, _cp, 0)

    def _stage_db(chunk, b):
        cb = chunk * G
        def _cp(j, inner):
            v = pk_v[pl.ds(cb + j * 16, 16)]
            db_r[b, pl.ds(j * 16, 16)] = lax.shift_right_logical(v, KBITS)
            return inner
        lax.fori_loop(0, G // 16, _cp, 0)

    def _fire(b):
        pltpu.async_copy(t_hbm.at[kb_r.at[b]], rows_r.at[b], sems[b])

    def _wait(b):
        pltpu.make_async_copy(t_hbm.at[kb_r.at[b]], rows_r.at[b], sems[b]).wait()

    _stage_kb(0, 0)
    _stage_db(0, 0)
    _fire(0)
    _stage_kb(1, 1)
    _stage_db(1, 1)
    _fire(1)

    def _main(g, carry):
        for b in range(2):
            k = 2 * g + b
            _wait(b)
            @pl.when(k + 2 < CH)
            def _():
                _stage_kb(k + 2, b)
            pltpu.sync_copy(rows_r.at[b], acc.at[db_r.at[b]], add=True)
            @pl.when(k + 2 < CH)
            def _():
                _fire(b)
                _stage_db(k + 2, b)
        return carry
    lax.fori_loop(0, CH // 2, _main, 0)
    plsc.subcore_barrier()

    rbase = s * 624
    pltpu.sync_copy(acc.at[pl.ds(rbase, 624)],
                    out_hbm.at[pl.ds(rbase, 624), pl.ds(c * H, H)])
    @pl.when(s == NSUB - 1)
    def _tail():
        pltpu.sync_copy(acc.at[pl.ds(9984, 16)],
                        out_hbm.at[pl.ds(9984, 16), pl.ds(c * H, H)])


def _final_body(agg_ref, norm_ref, lw_ref, out_ref):
    h2 = agg_ref[...] * norm_ref[...]
    out_ref[...] = jnp.maximum(
        h2 + jnp.dot(h2, lw_ref[...], preferred_element_type=jnp.float32), 0.0)


def _final(agg, normv, loop_weight):
    return pl.pallas_call(
        _final_body,
        grid=(NB,),
        in_specs=[
            pl.BlockSpec((BN, D), lambda i: (i, 0)),
            pl.BlockSpec((BN, 1), lambda i: (i, 0)),
            pl.BlockSpec((D, D), lambda i: (0, 0)),
        ],
        out_specs=pl.BlockSpec((BN, D), lambda i: (i, 0)),
        out_shape=jax.ShapeDtypeStruct((N, D), jnp.float32),
    )(agg, normv, loop_weight)


def kernel(x, norm, edge_index, edge_type, rFeatures, relation_weights, loop_weight):
    src = edge_index[0].astype(jnp.int32)
    dst = edge_index[1].astype(jnp.int32)
    key = edge_type.astype(jnp.int32) * N + src
    packed = lax.bitcast_convert_type(
        (dst.astype(jnp.uint32) << KBITS) | key.astype(jnp.uint32), jnp.int32)
    normv = norm.reshape(N, 1)

    sc_scatter = _make_sc_scatter()
    t0 = _transform(x, normv, relation_weights, rFeatures, apply_scale=False)
    agg1 = sc_scatter(packed, t0.reshape(2 * NR, H))
    t1 = _transform(agg1, normv, relation_weights, rFeatures, apply_scale=True)
    agg2 = sc_scatter(packed, t1.reshape(2 * NR, H))
    return _final(agg2, normv, loop_weight)
